# baseline (speedup 1.0000x reference)
import functools

import jax
import jax.numpy as jnp
from jax import lax
from jax.experimental import pallas as pl
from jax.experimental.pallas import tpu as pltpu

_N = 10000
_K = 32
_D = 128
_NPAD = 10240
_BLK = 256
_GRP = 8


def _linear_kernel(f_ref, w_ref, o_ref):
    o_ref[...] = lax.dot_general(
        f_ref[...], w_ref[...], (((1,), (0,)), ((), ())),
        preferred_element_type=jnp.float32)


def _attn_kernel(g_ref, vc_ref, vr_ref, b_ref, o_ref):
    G = g_ref[...]
    Vc = vc_ref[...]
    Vr = vr_ref[...]
    n_grp = _BLK // _GRP
    W_ = _GRP * _K
    bi = lax.broadcasted_iota(jnp.int32, (W_, W_), 0) // _K
    bj = lax.broadcasted_iota(jnp.int32, (W_, W_), 1) // _K
    blockmask = bi == bj
    eye = (lax.broadcasted_iota(jnp.int32, (W_, W_), 0)
           == lax.broadcasted_iota(jnp.int32, (W_, W_), 1)).astype(jnp.float32)
    dagg_rows = []
    for g in range(n_grp):
        X = G[g * W_:(g + 1) * W_, :]
        gram = lax.dot_general(
            X, X, (((1,), (1,)), ((), ())),
            preferred_element_type=jnp.float32)
        sq_c = jnp.sum(X * X, axis=1, keepdims=True)
        sq_r = lax.dot_general(
            sq_c, eye, (((0,), (0,)), ((), ())),
            precision=lax.Precision.HIGHEST,
            preferred_element_type=jnp.float32)
        v_c = Vc[g * W_:(g + 1) * W_] > 0
        v_r = Vr[g:g + 1, :] > 0
        d2 = jnp.maximum(sq_c + sq_r - 2.0 * gram, 0.0)
        dist = jnp.where(d2 > 0, jnp.sqrt(jnp.where(d2 > 0, d2, 1.0)), 0.0)
        dist = jnp.where(blockmask & v_c & v_r, dist, 0.0)
        dagg_rows.append(jnp.sum(dist, axis=0, keepdims=True))
    d_agg = jnp.concatenate(dagg_rows, axis=0)
    vmask = Vr > 0
    big = jnp.finfo(jnp.float32).max
    d_agg = jnp.where(vmask, d_agg, big)
    d_agg = jnp.where(jnp.isfinite(d_agg), d_agg, big)
    neg = -d_agg
    attn_segs = []
    for s in range(_GRP):
        seg = neg[:, s * _K:(s + 1) * _K]
        vseg = vmask[:, s * _K:(s + 1) * _K]
        m = jnp.max(seg, axis=1, keepdims=True)
        e = jnp.exp(seg - m)
        a = e / jnp.sum(e, axis=1, keepdims=True)
        a = a * vseg.astype(jnp.float32)
        a = a / jnp.sum(a, axis=1, keepdims=True)
        a = jnp.where(vseg, a, 0.0)
        attn_segs.append(a)
    attn = jnp.concatenate(attn_segs, axis=1)
    expand = (lax.broadcasted_iota(jnp.int32, (_GRP, W_), 1) // _K
              == lax.broadcasted_iota(jnp.int32, (_GRP, W_), 0)
              ).astype(jnp.float32)
    outs = []
    for g in range(n_grp):
        X = G[g * W_:(g + 1) * W_, :]
        a_mat = attn[g:g + 1, :] * expand
        outs.append(lax.dot_general(
            a_mat, X, (((1,), (0,)), ((), ())),
            preferred_element_type=jnp.float32))
    o_ref[...] = jnp.concatenate(outs, axis=0) + b_ref[...]


def _linear(feat, W):
    return pl.pallas_call(
        _linear_kernel,
        grid=(10,),
        in_specs=[pl.BlockSpec((1000, _D), lambda i: (i, 0)),
                  pl.BlockSpec((_D, _D), lambda i: (0, 0))],
        out_specs=pl.BlockSpec((1000, _D), lambda i: (i, 0)),
        out_shape=jax.ShapeDtypeStruct((_N, _D), jnp.float32),
    )(feat, W)


def _attention(gathered, vcol, vrow, b):
    nb = _NPAD // _BLK
    return pl.pallas_call(
        _attn_kernel,
        grid=(nb,),
        in_specs=[pl.BlockSpec((_BLK * _K, _D), lambda i: (i, 0)),
                  pl.BlockSpec((_BLK * _K, 1), lambda i: (i, 0)),
                  pl.BlockSpec((_BLK // _GRP, _GRP * _K), lambda i: (i, 0)),
                  pl.BlockSpec((1, _D), lambda i: (0, 0))],
        out_specs=pl.BlockSpec((_BLK, _D), lambda i: (i, 0)),
        out_shape=jax.ShapeDtypeStruct((_NPAD, _D), jnp.float32),
    )(gathered, vcol, vrow, b.reshape(1, _D))


def kernel(feat, edge_index, W, b):
    n = _N
    loops = jnp.arange(n, dtype=edge_index.dtype)
    rows = jnp.concatenate([edge_index[0], loops])
    cols = jnp.concatenate([edge_index[1], loops])
    h = _linear(feat, W)
    keys = rows * n + cols
    keys_s = jnp.sort(keys)
    rows_s = keys_s // n
    cols_s = keys_s % n
    deg = jnp.zeros((n,), jnp.int32).at[rows_s].add(1)
    row_start = jnp.cumsum(deg) - deg
    e = rows_s.shape[0]
    rank = jnp.arange(e, dtype=jnp.int32) - row_start[rows_s]
    topk = jnp.full((n, _K), -1, jnp.int32).at[rows_s, rank].set(
        cols_s, mode='drop')
    topk_p = jnp.concatenate(
        [topk, jnp.full((_NPAD - n, _K), -1, jnp.int32)], axis=0)
    safe = jnp.clip(topk_p, 0, n - 1)
    gathered = h[safe.reshape(-1)]
    valid = (topk_p != -1).astype(jnp.float32)
    vcol = valid.reshape(_NPAD * _K, 1)
    vrow = valid.reshape(_NPAD // _GRP, _GRP * _K)
    out = _attention(gathered, vcol, vrow, b)
    return out[:n]
